# baseline (speedup 1.0000x reference)
import functools

import jax
import jax.numpy as jnp
from jax import lax
from jax.experimental import pallas as pl
from jax.experimental.pallas import tpu as pltpu
from jax.experimental.pallas import tpu_sc as plsc

N_NODES = 10000
D = 128
K = 4
NC = 2
NS = 16
NW = NC * NS
NP = 2
RNG = 2560
CPD = 128
HPD = CPD // K
RING = 3
CH_A = 84
E_PAD = NW * CH_A * CPD
H_PAD = E_PAD // K
CH_B = H_PAD // (NS * CPD)
ACC_ROWS = 2688
ZSTRIPE = ACC_ROWS // NS
WSTRIPE = RNG // NS
DUMMY = RNG
NGA = (CH_A - RING) // RING
NGB = (CH_B - RING) // RING


def _sc_presum(x, src3):
    mesh = plsc.VectorSubcoreMesh(core_axis_name="c", subcore_axis_name="s")

    @functools.partial(
        pl.kernel,
        out_type=jax.ShapeDtypeStruct((H_PAD, D), jnp.float32),
        mesh=mesh,
        scratch_types=[
            pltpu.VMEM((CH_A, CPD), jnp.int32),
            pltpu.VMEM((CPD, D), jnp.float32),
            pltpu.VMEM((CPD, D), jnp.float32),
            pltpu.VMEM((CPD, D), jnp.float32),
            pltpu.VMEM((HPD, D), jnp.float32),
            pltpu.VMEM((HPD, D), jnp.float32),
            pltpu.VMEM((HPD, D), jnp.float32),
            pltpu.SemaphoreType.DMA,
            pltpu.SemaphoreType.DMA,
            pltpu.SemaphoreType.DMA,
            pltpu.SemaphoreType.DMA,
            pltpu.SemaphoreType.DMA,
            pltpu.SemaphoreType.DMA,
        ],
    )
    def ka(x_hbm, src_hbm, s_hbm,
           sidx, b0, b1, b2, sr0, sr1, sr2, s0, s1, s2, t0, t1, t2):
        bufs = (b0, b1, b2)
        gsem = (s0, s1, s2)
        srow = (sr0, sr1, sr2)
        ssem = (t0, t1, t2)
        cid = lax.axis_index("c")
        sid = lax.axis_index("s")
        wid = sid * NC + cid
        h0 = wid * CH_A * HPD

        pltpu.sync_copy(src_hbm.at[wid], sidx)

        def gstart(j, r):
            pltpu.async_copy(x_hbm.at[sidx.at[j]], bufs[r], gsem[r])

        def gwait(r):
            pltpu.make_async_copy(
                x_hbm.at[sidx.at[0]], bufs[r], gsem[r]).wait()

        def sum4(r, w):
            b = bufs[r]
            s = srow[w]

            def hbody(hh, carry):
                for dh in range(4):
                    h = hh * 4 + dh
                    for c in range(D // 16):
                        sl = pl.ds(c * 16, 16)
                        s[h, sl] = (b[4 * h, sl] + b[4 * h + 1, sl]) + (
                            b[4 * h + 2, sl] + b[4 * h + 3, sl])
                return carry

            lax.fori_loop(0, HPD // 4, hbody, 0)

        def step(j, r, first=False):
            gwait(r)
            if not first:
                pltpu.make_async_copy(
                    srow[r], s_hbm.at[pl.ds(h0, HPD)], ssem[r]).wait()
            sum4(r, r)
            pltpu.async_copy(
                srow[r], s_hbm.at[pl.ds(h0 + j * HPD, HPD)], ssem[r])

        for r in range(RING - 1):
            gstart(r, r)
        for r in range(RING):
            gstart(r + RING - 1, (r + RING - 1) % RING)
            step(r, r, first=True)

        def body(g, carry):
            for r in range(RING):
                j = g * RING + r
                gstart(j + RING - 1, (r + RING - 1) % RING)
                step(j, r)
            return carry

        lax.fori_loop(1, NGA, body, 0)

        base = NGA * RING
        gstart(CH_A - 1, (RING - 1) % RING)
        for r in range(RING):
            step(base + r, r)
        for r in range(RING):
            pltpu.make_async_copy(
                srow[r], s_hbm.at[pl.ds(h0, HPD)], ssem[r]).wait()

    return ka(x, src3)


def _sc_scatter(s, dst5, zrows):
    mesh = plsc.VectorSubcoreMesh(core_axis_name="c", subcore_axis_name="s")

    @functools.partial(
        pl.kernel,
        out_type=jax.ShapeDtypeStruct((NC * NP * RNG, D), jnp.float32),
        mesh=mesh,
        scratch_types=[
            pltpu.VMEM((CH_B, CPD), jnp.int32),
            pltpu.VMEM((CPD, D), jnp.float32),
            pltpu.VMEM((CPD, D), jnp.float32),
            pltpu.VMEM((CPD, D), jnp.float32),
            pltpu.VMEM_SHARED((ACC_ROWS, D), jnp.float32),
            pltpu.SemaphoreType.DMA,
            pltpu.SemaphoreType.DMA,
            pltpu.SemaphoreType.DMA,
        ],
    )
    def kb(s_hbm, dst_hbm, zr_hbm, out,
           didx, b0, b1, b2, acc, s0, s1, s2):
        bufs = (b0, b1, b2)
        gsem = (s0, s1, s2)
        cid = lax.axis_index("c")
        sid = lax.axis_index("s")
        row0 = sid * CH_B * CPD

        def gstart(j, r):
            pltpu.async_copy(
                s_hbm.at[pl.ds(row0 + j * CPD, CPD)], bufs[r], gsem[r])

        def gwait(r):
            pltpu.make_async_copy(
                s_hbm.at[pl.ds(row0, CPD)], bufs[r], gsem[r]).wait()

        def scat(j, r):
            pltpu.sync_copy(bufs[r], acc.at[didx.at[j]], add=True)

        for p in range(NP):
            pltpu.sync_copy(zr_hbm, acc.at[pl.ds(sid * ZSTRIPE, ZSTRIPE)])
            pltpu.sync_copy(dst_hbm.at[(cid * NP + p) * NS + sid], didx)
            plsc.subcore_barrier()

            for r in range(RING - 1):
                gstart(r, r)

            def body(g, carry):
                for r in range(RING):
                    j = g * RING + r
                    gwait(r)
                    scat(j, r)
                    gstart(j + RING - 1, (r + RING - 1) % RING)
                return carry

            lax.fori_loop(0, NGB, body, 0)

            base = NGB * RING
            for r in range(RING):
                gwait(r)
                scat(base + r, r)
                if base + r + RING - 1 < CH_B:
                    gstart(base + r + RING - 1, (r + RING - 1) % RING)

            plsc.subcore_barrier()
            out_base = (NP * cid + p) * RNG + sid * WSTRIPE
            pltpu.sync_copy(acc.at[pl.ds(sid * WSTRIPE, CPD)], b0)
            pltpu.sync_copy(b0, out.at[pl.ds(out_base, CPD)])
            rem = WSTRIPE - CPD
            pltpu.sync_copy(acc.at[pl.ds(sid * WSTRIPE + CPD, rem)],
                            b1.at[pl.ds(0, rem)])
            pltpu.sync_copy(b1.at[pl.ds(0, rem)],
                            out.at[pl.ds(out_base + CPD, rem)])
            plsc.subcore_barrier()

    return kb(s, dst5, zrows)


def _combine(x, p):
    def body(x_ref, p_ref, o_ref):
        o_ref[...] = x_ref[...] + p_ref[...]

    blk = 1000
    return pl.pallas_call(
        body,
        out_shape=jax.ShapeDtypeStruct((N_NODES, D), jnp.float32),
        grid=(N_NODES // blk,),
        in_specs=[pl.BlockSpec((blk, D), lambda i: (i, 0))] * 2,
        out_specs=pl.BlockSpec((blk, D), lambda i: (i, 0)),
    )(x, p)


def kernel(x, hyperedge_index):
    e = hyperedge_index.shape[1]
    src = hyperedge_index[0]
    dst = hyperedge_index[1].reshape(-1, K)[:, 0]
    src_p = jnp.concatenate([src, jnp.zeros((E_PAD - e,), jnp.int32)])
    dst_p = jnp.concatenate(
        [dst, jnp.full((H_PAD - e // K,), -1, jnp.int32)])
    src3 = src_p.reshape(NW, CH_A, CPD)
    base = (jnp.arange(NC * NP, dtype=jnp.int32) * RNG)[:, None]
    loc = dst_p[None, :] - base
    loc = jnp.where((loc >= 0) & (loc < RNG), loc, DUMMY)
    dst5 = loc.reshape(NC * NP * NS, CH_B, CPD)
    zrows = jnp.zeros((ZSTRIPE, D), jnp.float32)
    s = _sc_presum(x, src3)
    p = _sc_scatter(s, dst5, zrows)
    return _combine(x, p)


# speedup vs baseline: 1.0010x; 1.0010x over previous
import functools

import jax
import jax.numpy as jnp
from jax import lax
from jax.experimental import pallas as pl
from jax.experimental.pallas import tpu as pltpu
from jax.experimental.pallas import tpu_sc as plsc

N_NODES = 10000
D = 128
K = 4
NC = 2
NS = 16
NW = NC * NS
NP = 2
RNG = 2560
CPD = 128
HPD = CPD // K
RING = 3
CH_A = 84
E_PAD = NW * CH_A * CPD
H_PAD = E_PAD // K
CH_B = H_PAD // (NS * CPD)
ACC_ROWS = 2688
ZSTRIPE = ACC_ROWS // NS
WSTRIPE = RNG // NS
DUMMY = RNG
NGA = (CH_A - RING) // RING
NGB = (CH_B - RING) // RING


def _sc_presum(x, src3):
    mesh = plsc.VectorSubcoreMesh(core_axis_name="c", subcore_axis_name="s")

    @functools.partial(
        pl.kernel,
        out_type=jax.ShapeDtypeStruct((H_PAD, D), jnp.float32),
        mesh=mesh,
        scratch_types=[
            pltpu.VMEM((CH_A, CPD), jnp.int32),
            pltpu.VMEM((CPD, D), jnp.float32),
            pltpu.VMEM((CPD, D), jnp.float32),
            pltpu.VMEM((CPD, D), jnp.float32),
            pltpu.VMEM((HPD, D), jnp.float32),
            pltpu.VMEM((HPD, D), jnp.float32),
            pltpu.VMEM((HPD, D), jnp.float32),
            pltpu.SemaphoreType.DMA,
            pltpu.SemaphoreType.DMA,
            pltpu.SemaphoreType.DMA,
            pltpu.SemaphoreType.DMA,
            pltpu.SemaphoreType.DMA,
            pltpu.SemaphoreType.DMA,
        ],
    )
    def ka(x_hbm, src_hbm, s_hbm,
           sidx, b0, b1, b2, sr0, sr1, sr2, s0, s1, s2, t0, t1, t2):
        bufs = (b0, b1, b2)
        gsem = (s0, s1, s2)
        srow = (sr0, sr1, sr2)
        ssem = (t0, t1, t2)
        cid = lax.axis_index("c")
        sid = lax.axis_index("s")
        wid = sid * NC + cid
        h0 = wid * CH_A * HPD

        pltpu.sync_copy(src_hbm.at[wid], sidx)

        def gstart(j, r):
            pltpu.async_copy(x_hbm.at[sidx.at[j]], bufs[r], gsem[r])

        def gwait(r):
            pltpu.make_async_copy(
                x_hbm.at[sidx.at[0]], bufs[r], gsem[r]).wait()

        def sum4(r, w):
            b = bufs[r]
            s = srow[w]

            def hbody(h, carry):
                for c in range(D // 16):
                    sl = pl.ds(c * 16, 16)
                    s[h, sl] = (b[4 * h, sl] + b[4 * h + 1, sl]) + (
                        b[4 * h + 2, sl] + b[4 * h + 3, sl])
                return carry

            lax.fori_loop(0, HPD, hbody, 0)

        def step(j, r, first=False):
            gwait(r)
            if not first:
                pltpu.make_async_copy(
                    srow[r], s_hbm.at[pl.ds(h0, HPD)], ssem[r]).wait()
            sum4(r, r)
            pltpu.async_copy(
                srow[r], s_hbm.at[pl.ds(h0 + j * HPD, HPD)], ssem[r])

        for r in range(RING - 1):
            gstart(r, r)
        for r in range(RING):
            gstart(r + RING - 1, (r + RING - 1) % RING)
            step(r, r, first=True)

        def body(g, carry):
            for r in range(RING):
                j = g * RING + r
                gstart(j + RING - 1, (r + RING - 1) % RING)
                step(j, r)
            return carry

        lax.fori_loop(1, NGA, body, 0)

        base = NGA * RING
        gstart(CH_A - 1, (RING - 1) % RING)
        for r in range(RING):
            step(base + r, r)
        for r in range(RING):
            pltpu.make_async_copy(
                srow[r], s_hbm.at[pl.ds(h0, HPD)], ssem[r]).wait()

    return ka(x, src3)


def _sc_scatter(s, dst5, zrows):
    mesh = plsc.VectorSubcoreMesh(core_axis_name="c", subcore_axis_name="s")

    @functools.partial(
        pl.kernel,
        out_type=jax.ShapeDtypeStruct((NC * NP * RNG, D), jnp.float32),
        mesh=mesh,
        scratch_types=[
            pltpu.VMEM((CH_B, CPD), jnp.int32),
            pltpu.VMEM((CPD, D), jnp.float32),
            pltpu.VMEM((CPD, D), jnp.float32),
            pltpu.VMEM((CPD, D), jnp.float32),
            pltpu.VMEM_SHARED((ACC_ROWS, D), jnp.float32),
            pltpu.SemaphoreType.DMA,
            pltpu.SemaphoreType.DMA,
            pltpu.SemaphoreType.DMA,
        ],
    )
    def kb(s_hbm, dst_hbm, zr_hbm, out,
           didx, b0, b1, b2, acc, s0, s1, s2):
        bufs = (b0, b1, b2)
        gsem = (s0, s1, s2)
        cid = lax.axis_index("c")
        sid = lax.axis_index("s")
        row0 = sid * CH_B * CPD

        def gstart(j, r):
            pltpu.async_copy(
                s_hbm.at[pl.ds(row0 + j * CPD, CPD)], bufs[r], gsem[r])

        def gwait(r):
            pltpu.make_async_copy(
                s_hbm.at[pl.ds(row0, CPD)], bufs[r], gsem[r]).wait()

        def scat(j, r):
            pltpu.sync_copy(bufs[r], acc.at[didx.at[j]], add=True)

        for p in range(NP):
            pltpu.sync_copy(zr_hbm, acc.at[pl.ds(sid * ZSTRIPE, ZSTRIPE)])
            pltpu.sync_copy(dst_hbm.at[(cid * NP + p) * NS + sid], didx)
            plsc.subcore_barrier()

            for r in range(RING - 1):
                gstart(r, r)

            def body(g, carry):
                for r in range(RING):
                    j = g * RING + r
                    gwait(r)
                    scat(j, r)
                    gstart(j + RING - 1, (r + RING - 1) % RING)
                return carry

            lax.fori_loop(0, NGB, body, 0)

            base = NGB * RING
            for r in range(RING):
                gwait(r)
                scat(base + r, r)
                if base + r + RING - 1 < CH_B:
                    gstart(base + r + RING - 1, (r + RING - 1) % RING)

            plsc.subcore_barrier()
            out_base = (NP * cid + p) * RNG + sid * WSTRIPE
            pltpu.sync_copy(acc.at[pl.ds(sid * WSTRIPE, CPD)], b0)
            pltpu.sync_copy(b0, out.at[pl.ds(out_base, CPD)])
            rem = WSTRIPE - CPD
            pltpu.sync_copy(acc.at[pl.ds(sid * WSTRIPE + CPD, rem)],
                            b1.at[pl.ds(0, rem)])
            pltpu.sync_copy(b1.at[pl.ds(0, rem)],
                            out.at[pl.ds(out_base + CPD, rem)])
            plsc.subcore_barrier()

    return kb(s, dst5, zrows)


def _combine(x, p):
    def body(x_ref, p_ref, o_ref):
        o_ref[...] = x_ref[...] + p_ref[...]

    blk = 1000
    return pl.pallas_call(
        body,
        out_shape=jax.ShapeDtypeStruct((N_NODES, D), jnp.float32),
        grid=(N_NODES // blk,),
        in_specs=[pl.BlockSpec((blk, D), lambda i: (i, 0))] * 2,
        out_specs=pl.BlockSpec((blk, D), lambda i: (i, 0)),
    )(x, p)


def kernel(x, hyperedge_index):
    e = hyperedge_index.shape[1]
    src = hyperedge_index[0]
    dst = hyperedge_index[1].reshape(-1, K)[:, 0]
    src_p = jnp.concatenate([src, jnp.zeros((E_PAD - e,), jnp.int32)])
    dst_p = jnp.concatenate(
        [dst, jnp.full((H_PAD - e // K,), -1, jnp.int32)])
    src3 = src_p.reshape(NW, CH_A, CPD)
    base = (jnp.arange(NC * NP, dtype=jnp.int32) * RNG)[:, None]
    loc = dst_p[None, :] - base
    loc = jnp.where((loc >= 0) & (loc < RNG), loc, DUMMY)
    dst5 = loc.reshape(NC * NP * NS, CH_B, CPD)
    zrows = jnp.zeros((ZSTRIPE, D), jnp.float32)
    s = _sc_presum(x, src3)
    p = _sc_scatter(s, dst5, zrows)
    return _combine(x, p)


# speedup vs baseline: 1.0032x; 1.0022x over previous
import functools

import jax
import jax.numpy as jnp
from jax import lax
from jax.experimental import pallas as pl
from jax.experimental.pallas import tpu as pltpu
from jax.experimental.pallas import tpu_sc as plsc

N_NODES = 10000
D = 128
K = 4
NC = 2
NS = 16
NW = NC * NS
NP = 2
RNG = 2560
CPD = 128
HPD = CPD // K
RING = 3
CH_A = 84
E_PAD = NW * CH_A * CPD
H_PAD = E_PAD // K
CH_B = H_PAD // (NS * CPD)
ACC_ROWS = 2688
ZSTRIPE = ACC_ROWS // NS
WSTRIPE = RNG // NS
DUMMY = RNG
RING_A = 4
NGA = (CH_A - RING_A) // RING_A
NGB = (CH_B - RING) // RING


def _sc_presum(x, src3):
    mesh = plsc.VectorSubcoreMesh(core_axis_name="c", subcore_axis_name="s")

    @functools.partial(
        pl.kernel,
        out_type=jax.ShapeDtypeStruct((H_PAD, D), jnp.float32),
        mesh=mesh,
        scratch_types=[
            pltpu.VMEM((CH_A, CPD), jnp.int32),
            pltpu.VMEM((CPD, D), jnp.float32),
            pltpu.VMEM((CPD, D), jnp.float32),
            pltpu.VMEM((CPD, D), jnp.float32),
            pltpu.VMEM((CPD, D), jnp.float32),
            pltpu.VMEM((HPD, D), jnp.float32),
            pltpu.VMEM((HPD, D), jnp.float32),
            pltpu.VMEM((HPD, D), jnp.float32),
            pltpu.VMEM((HPD, D), jnp.float32),
            pltpu.SemaphoreType.DMA,
            pltpu.SemaphoreType.DMA,
            pltpu.SemaphoreType.DMA,
            pltpu.SemaphoreType.DMA,
            pltpu.SemaphoreType.DMA,
            pltpu.SemaphoreType.DMA,
            pltpu.SemaphoreType.DMA,
            pltpu.SemaphoreType.DMA,
        ],
    )
    def ka(x_hbm, src_hbm, s_hbm,
           sidx, b0, b1, b2, b3, sr0, sr1, sr2, sr3,
           s0, s1, s2, s3, t0, t1, t2, t3):
        bufs = (b0, b1, b2, b3)
        gsem = (s0, s1, s2, s3)
        srow = (sr0, sr1, sr2, sr3)
        ssem = (t0, t1, t2, t3)
        cid = lax.axis_index("c")
        sid = lax.axis_index("s")
        wid = sid * NC + cid
        h0 = wid * CH_A * HPD

        pltpu.sync_copy(src_hbm.at[wid], sidx)

        def gstart(j, r):
            pltpu.async_copy(x_hbm.at[sidx.at[j]], bufs[r], gsem[r])

        def gwait(r):
            pltpu.make_async_copy(
                x_hbm.at[sidx.at[0]], bufs[r], gsem[r]).wait()

        def sum4(r, w):
            b = bufs[r]
            s = srow[w]

            def hbody(h, carry):
                for c in range(D // 16):
                    sl = pl.ds(c * 16, 16)
                    s[h, sl] = (b[4 * h, sl] + b[4 * h + 1, sl]) + (
                        b[4 * h + 2, sl] + b[4 * h + 3, sl])
                return carry

            lax.fori_loop(0, HPD, hbody, 0)

        def step(j, r, first=False):
            gwait(r)
            if not first:
                pltpu.make_async_copy(
                    srow[r], s_hbm.at[pl.ds(h0, HPD)], ssem[r]).wait()
            sum4(r, r)
            pltpu.async_copy(
                srow[r], s_hbm.at[pl.ds(h0 + j * HPD, HPD)], ssem[r])

        for r in range(RING_A - 1):
            gstart(r, r)
        for r in range(RING_A):
            gstart(r + RING_A - 1, (r + RING_A - 1) % RING_A)
            step(r, r, first=True)

        def body(g, carry):
            for r in range(RING_A):
                j = g * RING_A + r
                gstart(j + RING_A - 1, (r + RING_A - 1) % RING_A)
                step(j, r)
            return carry

        lax.fori_loop(1, NGA, body, 0)

        base = NGA * RING_A
        gstart(CH_A - 1, (RING_A - 1) % RING_A)
        for r in range(RING_A):
            step(base + r, r)
        for r in range(RING_A):
            pltpu.make_async_copy(
                srow[r], s_hbm.at[pl.ds(h0, HPD)], ssem[r]).wait()

    return ka(x, src3)


def _sc_scatter(s, dst5, zrows):
    mesh = plsc.VectorSubcoreMesh(core_axis_name="c", subcore_axis_name="s")

    @functools.partial(
        pl.kernel,
        out_type=jax.ShapeDtypeStruct((NC * NP * RNG, D), jnp.float32),
        mesh=mesh,
        scratch_types=[
            pltpu.VMEM((CH_B, CPD), jnp.int32),
            pltpu.VMEM((CPD, D), jnp.float32),
            pltpu.VMEM((CPD, D), jnp.float32),
            pltpu.VMEM((CPD, D), jnp.float32),
            pltpu.VMEM_SHARED((ACC_ROWS, D), jnp.float32),
            pltpu.SemaphoreType.DMA,
            pltpu.SemaphoreType.DMA,
            pltpu.SemaphoreType.DMA,
        ],
    )
    def kb(s_hbm, dst_hbm, zr_hbm, out,
           didx, b0, b1, b2, acc, s0, s1, s2):
        bufs = (b0, b1, b2)
        gsem = (s0, s1, s2)
        cid = lax.axis_index("c")
        sid = lax.axis_index("s")
        row0 = sid * CH_B * CPD

        def gstart(j, r):
            pltpu.async_copy(
                s_hbm.at[pl.ds(row0 + j * CPD, CPD)], bufs[r], gsem[r])

        def gwait(r):
            pltpu.make_async_copy(
                s_hbm.at[pl.ds(row0, CPD)], bufs[r], gsem[r]).wait()

        def scat(j, r):
            pltpu.sync_copy(bufs[r], acc.at[didx.at[j]], add=True)

        for p in range(NP):
            pltpu.sync_copy(zr_hbm, acc.at[pl.ds(sid * ZSTRIPE, ZSTRIPE)])
            pltpu.sync_copy(dst_hbm.at[(cid * NP + p) * NS + sid], didx)
            plsc.subcore_barrier()

            for r in range(RING - 1):
                gstart(r, r)

            def body(g, carry):
                for r in range(RING):
                    j = g * RING + r
                    gwait(r)
                    scat(j, r)
                    gstart(j + RING - 1, (r + RING - 1) % RING)
                return carry

            lax.fori_loop(0, NGB, body, 0)

            base = NGB * RING
            for r in range(RING):
                gwait(r)
                scat(base + r, r)
                if base + r + RING - 1 < CH_B:
                    gstart(base + r + RING - 1, (r + RING - 1) % RING)

            plsc.subcore_barrier()
            out_base = (NP * cid + p) * RNG + sid * WSTRIPE
            pltpu.sync_copy(acc.at[pl.ds(sid * WSTRIPE, CPD)], b0)
            pltpu.sync_copy(b0, out.at[pl.ds(out_base, CPD)])
            rem = WSTRIPE - CPD
            pltpu.sync_copy(acc.at[pl.ds(sid * WSTRIPE + CPD, rem)],
                            b1.at[pl.ds(0, rem)])
            pltpu.sync_copy(b1.at[pl.ds(0, rem)],
                            out.at[pl.ds(out_base + CPD, rem)])
            plsc.subcore_barrier()

    return kb(s, dst5, zrows)


def _combine(x, p):
    def body(x_ref, p_ref, o_ref):
        o_ref[...] = x_ref[...] + p_ref[...]

    blk = 1000
    return pl.pallas_call(
        body,
        out_shape=jax.ShapeDtypeStruct((N_NODES, D), jnp.float32),
        grid=(N_NODES // blk,),
        in_specs=[pl.BlockSpec((blk, D), lambda i: (i, 0))] * 2,
        out_specs=pl.BlockSpec((blk, D), lambda i: (i, 0)),
    )(x, p)


def kernel(x, hyperedge_index):
    e = hyperedge_index.shape[1]
    src = hyperedge_index[0]
    dst = hyperedge_index[1].reshape(-1, K)[:, 0]
    src_p = jnp.concatenate([src, jnp.zeros((E_PAD - e,), jnp.int32)])
    dst_p = jnp.concatenate(
        [dst, jnp.full((H_PAD - e // K,), -1, jnp.int32)])
    src3 = src_p.reshape(NW, CH_A, CPD)
    base = (jnp.arange(NC * NP, dtype=jnp.int32) * RNG)[:, None]
    loc = dst_p[None, :] - base
    loc = jnp.where((loc >= 0) & (loc < RNG), loc, DUMMY)
    dst5 = loc.reshape(NC * NP * NS, CH_B, CPD)
    zrows = jnp.zeros((ZSTRIPE, D), jnp.float32)
    s = _sc_presum(x, src3)
    p = _sc_scatter(s, dst5, zrows)
    return _combine(x, p)
